# baseline (speedup 1.0000x reference)
import functools

import jax
import jax.numpy as jnp
from jax import lax
from jax.experimental import pallas as pl
from jax.experimental.pallas import tpu as pltpu
from jax.experimental.pallas import tpu_sc as plsc

N_NODES = 10000
N_ACTORS = 2000
D = 128
NUM_MODS = 6
NUM_PREDS = 30

NC = 2
NS = 16
NW = NC * NS
L = 16
K = 128


def _ceil_div(a, b):
    return (a + b - 1) // b


def _rpt(n_dst):
    return _ceil_div(_ceil_div(n_dst, NS), 8) * 8


def _zero_rows_buf(rows):
    zero = jnp.zeros((L,), jnp.float32)

    def body(r, _):
        for c in range(D // L):
            rows[r, pl.ds(c * L, L)] = zero
        return ()

    lax.fori_loop(0, K, body, ())


def _zero_acc(acc, rows, rpt, sid):
    base = sid * rpt
    for j in range(_ceil_div(rpt, K)):
        sz = min(K, rpt - j * K)
        pltpu.sync_copy(rows.at[pl.ds(0, sz)], acc.at[pl.ds(base + j * K, sz)])


def _writeout(acc, out, rpt, cid, sid):
    base = sid * rpt
    for j in range(_ceil_div(rpt, K)):
        sz = min(K, rpt - j * K)
        pltpu.sync_copy(acc.at[pl.ds(base + j * K, sz)],
                        out.at[cid, pl.ds(base + j * K, sz)])


def _make_seg_sum(n_dst, npt, S=8):
    rpt = _rpt(n_dst)
    n_pad = NS * rpt
    G = npt // S
    assert npt % S == 0 and S % 8 == 0
    mesh = plsc.VectorSubcoreMesh(core_axis_name="c", subcore_axis_name="s",
                                  num_cores=NC, num_subcores=NS)

    @functools.partial(
        pl.kernel,
        out_type=jax.ShapeDtypeStruct((NC, n_pad, D), jnp.float32),
        mesh=mesh,
        scratch_types=[
            pltpu.VMEM_SHARED((n_pad, D), jnp.float32),
            pltpu.VMEM((S, K), jnp.int32),
            pltpu.VMEM((S, K), jnp.int32),
            pltpu.VMEM((K, D), jnp.float32),
            pltpu.SemaphoreType.DMA,
        ],
    )
    def k(table, src, dst, out, acc, slab_s, slab_d, rows, sem):
        cid = lax.axis_index("c")
        sid = lax.axis_index("s")
        wid = sid * NC + cid
        base0 = wid * npt

        _zero_rows_buf(rows)
        _zero_acc(acc, rows, rpt, sid)
        plsc.subcore_barrier()

        def group(g, _):
            gb = base0 + g * S
            pltpu.sync_copy(src.at[pl.ds(gb, S)], slab_s)
            pltpu.sync_copy(dst.at[pl.ds(gb, S)], slab_d)
            for b in range(S):
                pltpu.async_copy(table.at[slab_s.at[b]], rows, sem).wait()
                pltpu.sync_copy(rows, acc.at[slab_d.at[b]], add=True)
            return ()

        lax.fori_loop(0, G, group, ())
        plsc.subcore_barrier()
        _writeout(acc, out, rpt, cid, sid)

    return k


def _make_relu_agg(n_dst, npt, S=8):
    rpt = _rpt(n_dst)
    n_pad = NS * rpt
    G = npt // S
    assert npt % S == 0 and S % 8 == 0
    mesh = plsc.VectorSubcoreMesh(core_axis_name="c", subcore_axis_name="s",
                                  num_cores=NC, num_subcores=NS)

    @functools.partial(
        pl.kernel,
        out_type=jax.ShapeDtypeStruct((NC, n_pad, D), jnp.float32),
        mesh=mesh,
        scratch_types=[
            pltpu.VMEM_SHARED((n_pad, D), jnp.float32),
            pltpu.VMEM((S, K), jnp.int32),
            pltpu.VMEM((S, K), jnp.int32),
            pltpu.VMEM((K, D), jnp.float32),
            pltpu.VMEM((K, D), jnp.float32),
            pltpu.SemaphoreType.DMA,
            pltpu.SemaphoreType.DMA,
        ],
    )
    def k(s_tab, q_tab, src, dst, out, acc, slab_s, slab_d, rows_a, rows_b,
          sem_a, sem_b):
        cid = lax.axis_index("c")
        sid = lax.axis_index("s")
        wid = sid * NC + cid
        base0 = wid * npt

        _zero_rows_buf(rows_a)
        _zero_acc(acc, rows_a, rpt, sid)
        plsc.subcore_barrier()

        def group(g, _):
            gb = base0 + g * S
            pltpu.sync_copy(src.at[pl.ds(gb, S)], slab_s)
            pltpu.sync_copy(dst.at[pl.ds(gb, S)], slab_d)
            for b in range(S):
                cp_a = pltpu.async_copy(s_tab.at[slab_s.at[b]], rows_a, sem_a)
                cp_b = pltpu.async_copy(q_tab.at[slab_d.at[b]], rows_b, sem_b)
                cp_a.wait()
                cp_b.wait()

                def rbody(r, _):
                    for c in range(D // L):
                        cs = pl.ds(c * L, L)
                        va = rows_a[r, cs]
                        vb = rows_b[r, cs]
                        rows_a[r, cs] = jnp.maximum(va + vb, 0.0)
                    return ()

                lax.fori_loop(0, K, rbody, ())
                pltpu.sync_copy(rows_a, acc.at[slab_d.at[b]], add=True)
            return ()

        lax.fori_loop(0, G, group, ())
        plsc.subcore_barrier()
        _writeout(acc, out, rpt, cid, sid)

    return k


_NPT_SEG = 80
_NPT_A2M = 40
_NPT_A2A = 16

_seg_sum_nodes = _make_seg_sum(N_NODES, _NPT_SEG)
_relu_agg_nodes = _make_relu_agg(N_NODES, _NPT_A2M)
_relu_agg_actors = _make_relu_agg(N_ACTORS, _NPT_A2M)
_relu_agg_a2a = _make_relu_agg(N_ACTORS, _NPT_A2A)


def _pad_edges(src, dst, npt, n_dst):
    e_pad = (NW * npt + 16) * K - src.shape[0]
    src_p = jnp.concatenate([src, jnp.zeros((e_pad,), jnp.int32)])
    dst_p = jnp.concatenate([dst, jnp.full((e_pad,), n_dst, jnp.int32)])
    return src_p.reshape(-1, K), dst_p.reshape(-1, K)


def _proj_kernel(x_ref, c_ref, wt_ref, wb_ref, o_ref):
    x = x_ref[...]
    c = c_ref[...]
    o = jnp.dot(x, wt_ref[...], preferred_element_type=jnp.float32)
    o -= c[:, 0:1] * wb_ref[0:1, :] + c[:, 1:2] * wb_ref[1:2, :]
    o_ref[...] = o


def _proj(x, ctrs, w):
    n = x.shape[0]
    bn = 1000 if n % 1000 == 0 else n
    grid = n // bn
    return pl.pallas_call(
        _proj_kernel,
        grid=(grid,),
        in_specs=[
            pl.BlockSpec((bn, D), lambda i: (i, 0)),
            pl.BlockSpec((bn, 2), lambda i: (i, 0)),
            pl.BlockSpec((D, D), lambda i: (0, 0)),
            pl.BlockSpec((2, D), lambda i: (0, 0)),
        ],
        out_specs=pl.BlockSpec((bn, D), lambda i: (i, 0)),
        out_shape=jax.ShapeDtypeStruct((n, D), jnp.float32),
    )(x, ctrs, w[:D], w[D:])


def _ctrproj_kernel(c_ref, wb_ref, o_ref):
    c = c_ref[...]
    o_ref[...] = c[:, 0:1] * wb_ref[0:1, :] + c[:, 1:2] * wb_ref[1:2, :]


def _ctrproj(ctrs, wb):
    n = ctrs.shape[0]
    bn = 1000 if n % 1000 == 0 else n
    grid = n // bn
    return pl.pallas_call(
        _ctrproj_kernel,
        grid=(grid,),
        in_specs=[
            pl.BlockSpec((bn, 2), lambda i: (i, 0)),
            pl.BlockSpec((2, D), lambda i: (0, 0)),
        ],
        out_specs=pl.BlockSpec((bn, D), lambda i: (i, 0)),
        out_shape=jax.ShapeDtypeStruct((n, D), jnp.float32),
    )(ctrs, wb)


def _map_update_kernel(n_ref, s0_ref, s1_ref, w1_ref, w2_ref, o_ref):
    o = jnp.dot(n_ref[...], w1_ref[...], preferred_element_type=jnp.float32)
    o += jnp.dot(s0_ref[...] + s1_ref[...], w2_ref[...],
                 preferred_element_type=jnp.float32)
    o_ref[...] = jnp.maximum(o, 0.0)


def _map_update(nodes, seg, w1, w2):
    n = nodes.shape[0]
    bn = 1000
    return pl.pallas_call(
        _map_update_kernel,
        grid=(n // bn,),
        in_specs=[
            pl.BlockSpec((bn, D), lambda i: (i, 0)),
            pl.BlockSpec((bn, D), lambda i: (i, 0)),
            pl.BlockSpec((bn, D), lambda i: (i, 0)),
            pl.BlockSpec((D, D), lambda i: (0, 0)),
            pl.BlockSpec((D, D), lambda i: (0, 0)),
        ],
        out_specs=pl.BlockSpec((bn, D), lambda i: (i, 0)),
        out_shape=jax.ShapeDtypeStruct((n, D), jnp.float32),
    )(nodes, seg[0], seg[1], w1, w2)


def _map_update2_kernel(n_ref, s0_ref, s1_ref, a0_ref, a1_ref, w1_ref, w2_ref,
                        o_ref):
    o = jnp.dot(n_ref[...], w1_ref[...], preferred_element_type=jnp.float32)
    o += jnp.dot(s0_ref[...] + s1_ref[...], w2_ref[...],
                 preferred_element_type=jnp.float32)
    o_ref[...] = jnp.maximum(o, 0.0) + a0_ref[...] + a1_ref[...]


def _map_update2(nodes, seg, a2m, w1, w2):
    n = nodes.shape[0]
    bn = 1000
    return pl.pallas_call(
        _map_update2_kernel,
        grid=(n // bn,),
        in_specs=[pl.BlockSpec((bn, D), lambda i: (i, 0))] * 5 + [
            pl.BlockSpec((D, D), lambda i: (0, 0)),
            pl.BlockSpec((D, D), lambda i: (0, 0)),
        ],
        out_specs=pl.BlockSpec((bn, D), lambda i: (i, 0)),
        out_shape=jax.ShapeDtypeStruct((n, D), jnp.float32),
    )(nodes, seg[0], seg[1], a2m[0], a2m[1], w1, w2)


def _pred_kernel(a_ref, p0_ref, p1_ref, wp_ref, bp_ref, o_ref):
    a5 = jnp.maximum(a_ref[...] + p0_ref[...] + p1_ref[...], 0.0)
    o_ref[...] = (jnp.dot(a5, wp_ref[...], preferred_element_type=jnp.float32)
                  + bp_ref[...])


def _pred(actors4, a2a, wp, bp):
    n = actors4.shape[0]
    m = wp.shape[1]
    return pl.pallas_call(
        _pred_kernel,
        grid=(1,),
        in_specs=[
            pl.BlockSpec((n, D), lambda i: (0, 0)),
            pl.BlockSpec((n, D), lambda i: (0, 0)),
            pl.BlockSpec((n, D), lambda i: (0, 0)),
            pl.BlockSpec((D, m), lambda i: (0, 0)),
            pl.BlockSpec((1, m), lambda i: (0, 0)),
        ],
        out_specs=pl.BlockSpec((n, m), lambda i: (0, 0)),
        out_shape=jax.ShapeDtypeStruct((n, m), jnp.float32),
    )(actors4, a2a[0], a2a[1], wp, bp)


def kernel(actors, actor_ctrs, nodes, node_ctrs, edge_index, a2m_src, a2m_dst,
           m2a_src, m2a_dst, a2a_index, rot, orig,
           W_map1, W_map2, W_a2m, W_m2a, W_a2a, W_pred, b_pred):
    src, dst = _pad_edges(edge_index[0], edge_index[1], _NPT_SEG, N_NODES)
    a2m_s, a2m_d = _pad_edges(a2m_src, a2m_dst, _NPT_A2M, N_NODES)
    m2a_s, m2a_d = _pad_edges(m2a_src, m2a_dst, _NPT_A2M, N_ACTORS)
    a2a_s, a2a_d = _pad_edges(a2a_index[0], a2a_index[1], _NPT_A2A, N_ACTORS)

    npad_n = NS * _rpt(N_NODES)
    npad_a = NS * _rpt(N_ACTORS)
    node_ctrs_p = jnp.concatenate(
        [node_ctrs, jnp.zeros((npad_n - N_NODES, 2), jnp.float32)])
    actor_ctrs_p = jnp.concatenate(
        [actor_ctrs, jnp.zeros((npad_a - N_ACTORS, 2), jnp.float32)])

    S_a = _proj(actors, actor_ctrs, W_a2m)
    Q_n = _ctrproj(node_ctrs_p, W_a2m[D:])

    seg1 = _seg_sum_nodes(nodes, src, dst)[:, :N_NODES]
    a2m_agg = _relu_agg_nodes(S_a, Q_n, a2m_s, a2m_d)[:, :N_NODES]
    nodes2 = _map_update2(nodes, seg1, a2m_agg, W_map1, W_map2)

    seg2 = _seg_sum_nodes(nodes2, src, dst)[:, :N_NODES]
    nodes3 = _map_update(nodes2, seg2, W_map1, W_map2)

    S_n = _proj(nodes3, node_ctrs, W_m2a)
    Q_a = _ctrproj(actor_ctrs_p, W_m2a[D:])
    m2a_agg = _relu_agg_actors(S_n, Q_a, m2a_s, m2a_d)[:, :N_ACTORS]
    actors4 = actors + m2a_agg[0] + m2a_agg[1]

    S_aa = _proj(actors4, actor_ctrs, W_a2a)
    Q_aa = _ctrproj(actor_ctrs_p, W_a2a[D:])
    a2a_agg = _relu_agg_a2a(S_aa, Q_aa, a2a_s, a2a_d)[:, :N_ACTORS]

    wp = (W_pred.reshape(D, -1, 2) @ rot).reshape(D, -1)
    bp = (b_pred.reshape(-1, 2) @ rot + orig[None, :]).reshape(1, -1)
    reg = _pred(actors4, a2a_agg, wp, bp)
    return reg.reshape(N_ACTORS, NUM_MODS, NUM_PREDS, 2)


# speedup vs baseline: 2.3346x; 2.3346x over previous
import functools

import jax
import jax.numpy as jnp
from jax import lax
from jax.experimental import pallas as pl
from jax.experimental.pallas import tpu as pltpu
from jax.experimental.pallas import tpu_sc as plsc

N_NODES = 10000
N_ACTORS = 2000
D = 128
NUM_MODS = 6
NUM_PREDS = 30

NC = 2
NS = 16
NW = NC * NS
L = 16
K = 128


def _ceil_div(a, b):
    return (a + b - 1) // b


def _rpt(n_dst):
    return _ceil_div(_ceil_div(n_dst, NS), 8) * 8


def _zero_rows_buf(rows):
    zero = jnp.zeros((L,), jnp.float32)

    def body(r, _):
        for c in range(D // L):
            rows[r, pl.ds(c * L, L)] = zero
        return ()

    lax.fori_loop(0, K, body, ())


def _zero_acc(acc, rows, rpt, sid):
    base = sid * rpt
    for j in range(_ceil_div(rpt, K)):
        sz = min(K, rpt - j * K)
        pltpu.sync_copy(rows.at[pl.ds(0, sz)], acc.at[pl.ds(base + j * K, sz)])


def _writeout(acc, out, rpt, cid, sid):
    base = sid * rpt
    for j in range(_ceil_div(rpt, K)):
        sz = min(K, rpt - j * K)
        pltpu.sync_copy(acc.at[pl.ds(base + j * K, sz)],
                        out.at[cid, pl.ds(base + j * K, sz)])


def _make_seg_sum(n_dst, n_edges, M=2):
    rpt = _rpt(n_dst)
    n_pad = NS * rpt
    nsc = n_edges // (M * K)
    assert n_edges % (M * K) == 0
    mesh = plsc.VectorSubcoreMesh(core_axis_name="c", subcore_axis_name="s",
                                  num_cores=NC, num_subcores=NS)

    @functools.partial(
        pl.kernel,
        out_type=jax.ShapeDtypeStruct((NC, n_pad, D), jnp.float32),
        mesh=mesh,
        scratch_types=(
            [pltpu.VMEM_SHARED((n_pad, D), jnp.float32),
             pltpu.VMEM((M * K,), jnp.int32)]
            + [pltpu.VMEM((K,), jnp.int32)] * M
            + [pltpu.VMEM((M * K, D), jnp.float32)]
            + [pltpu.SemaphoreType.DMA] * (3 * M)
        ),
    )
    def k(table, src, dst, out, acc, idx_s, *rest):
        idx_d = rest[:M]
        rows = rest[M]
        dsem = rest[M + 1:M + 1 + M]
        gsem = rest[M + 1 + M:M + 1 + 2 * M]
        ssem = rest[M + 1 + 2 * M:M + 1 + 3 * M]
        cid = lax.axis_index("c")
        sid = lax.axis_index("s")
        wid = sid * NC + cid

        _zero_rows_buf(rows)
        _zero_acc(acc, rows, rpt, sid)
        plsc.subcore_barrier()

        nloops = (nsc - wid + NW - 1) // NW

        def body(j, _):
            base = (wid + j * NW) * (M * K)
            for m in range(M):
                pltpu.async_copy(dst.at[pl.ds(base + m * K, K)], idx_d[m],
                                 dsem[m])
            pltpu.sync_copy(src.at[pl.ds(base, M * K)], idx_s)
            for m in range(M):
                pltpu.make_async_copy(dst.at[pl.ds(base + m * K, K)],
                                      idx_d[m], dsem[m]).wait()
            for m in range(M):
                pltpu.async_copy(table.at[idx_s.at[pl.ds(m * K, K)]],
                                 rows.at[pl.ds(m * K, K)], gsem[m])
            for m in range(M):
                pltpu.make_async_copy(table.at[idx_s.at[pl.ds(m * K, K)]],
                                      rows.at[pl.ds(m * K, K)], gsem[m]).wait()
            for m in range(M):
                pltpu.async_copy(rows.at[pl.ds(m * K, K)], acc.at[idx_d[m]],
                                 ssem[m], add=True)
            for m in range(M):
                pltpu.make_async_copy(rows.at[pl.ds(m * K, K)],
                                      acc.at[idx_d[m]], ssem[m]).wait()
            return ()

        lax.fori_loop(0, nloops, body, ())
        plsc.subcore_barrier()
        _writeout(acc, out, rpt, cid, sid)

    return k


def _make_relu_agg(n_dst, n_edges, M=1):
    rpt = _rpt(n_dst)
    n_pad = NS * rpt
    nsc = n_edges // (M * K)
    assert n_edges % (M * K) == 0
    mesh = plsc.VectorSubcoreMesh(core_axis_name="c", subcore_axis_name="s",
                                  num_cores=NC, num_subcores=NS)

    @functools.partial(
        pl.kernel,
        out_type=jax.ShapeDtypeStruct((NC, n_pad, D), jnp.float32),
        mesh=mesh,
        scratch_types=(
            [pltpu.VMEM_SHARED((n_pad, D), jnp.float32),
             pltpu.VMEM((M * K,), jnp.int32)]
            + [pltpu.VMEM((K,), jnp.int32)] * M
            + [pltpu.VMEM((M * K, D), jnp.float32),
               pltpu.VMEM((M * K, D), jnp.float32)]
            + [pltpu.SemaphoreType.DMA] * (4 * M)
        ),
    )
    def k(s_tab, q_tab, src, dst, out, acc, idx_s, *rest):
        idx_d = rest[:M]
        rows_a = rest[M]
        rows_b = rest[M + 1]
        dsem = rest[M + 2:M + 2 + M]
        gsem_a = rest[M + 2 + M:M + 2 + 2 * M]
        gsem_b = rest[M + 2 + 2 * M:M + 2 + 3 * M]
        ssem = rest[M + 2 + 3 * M:M + 2 + 4 * M]
        cid = lax.axis_index("c")
        sid = lax.axis_index("s")
        wid = sid * NC + cid

        _zero_rows_buf(rows_a)
        _zero_acc(acc, rows_a, rpt, sid)
        plsc.subcore_barrier()

        nloops = (nsc - wid + NW - 1) // NW

        def body(j, _):
            base = (wid + j * NW) * (M * K)
            for m in range(M):
                pltpu.async_copy(dst.at[pl.ds(base + m * K, K)], idx_d[m],
                                 dsem[m])
            pltpu.sync_copy(src.at[pl.ds(base, M * K)], idx_s)
            for m in range(M):
                pltpu.make_async_copy(dst.at[pl.ds(base + m * K, K)],
                                      idx_d[m], dsem[m]).wait()
            for m in range(M):
                pltpu.async_copy(s_tab.at[idx_s.at[pl.ds(m * K, K)]],
                                 rows_a.at[pl.ds(m * K, K)], gsem_a[m])
                pltpu.async_copy(q_tab.at[idx_d[m]],
                                 rows_b.at[pl.ds(m * K, K)], gsem_b[m])
            for m in range(M):
                pltpu.make_async_copy(s_tab.at[idx_s.at[pl.ds(m * K, K)]],
                                      rows_a.at[pl.ds(m * K, K)],
                                      gsem_a[m]).wait()
                pltpu.make_async_copy(q_tab.at[idx_d[m]],
                                      rows_b.at[pl.ds(m * K, K)],
                                      gsem_b[m]).wait()

            def rbody(r, _):
                for c in range(D // L):
                    cs = pl.ds(c * L, L)
                    va = rows_a[r, cs]
                    vb = rows_b[r, cs]
                    rows_a[r, cs] = jnp.maximum(va + vb, 0.0)
                return ()

            lax.fori_loop(0, M * K, rbody, ())
            for m in range(M):
                pltpu.async_copy(rows_a.at[pl.ds(m * K, K)], acc.at[idx_d[m]],
                                 ssem[m], add=True)
            for m in range(M):
                pltpu.make_async_copy(rows_a.at[pl.ds(m * K, K)],
                                      acc.at[idx_d[m]], ssem[m]).wait()
            return ()

        lax.fori_loop(0, nloops, body, ())
        plsc.subcore_barrier()
        _writeout(acc, out, rpt, cid, sid)

    return k


_seg_sum_nodes = _make_seg_sum(N_NODES, 320000, M=2)
_relu_agg_nodes = _make_relu_agg(N_NODES, 160000, M=1)
_relu_agg_actors = _make_relu_agg(N_ACTORS, 160000, M=2)
_relu_agg_a2a = _make_relu_agg(N_ACTORS, 64000, M=2)


def _proj_kernel(x_ref, c_ref, wt_ref, wb_ref, o_ref):
    x = x_ref[...]
    c = c_ref[...]
    o = jnp.dot(x, wt_ref[...], preferred_element_type=jnp.float32)
    o -= c[:, 0:1] * wb_ref[0:1, :] + c[:, 1:2] * wb_ref[1:2, :]
    o_ref[...] = o


def _proj(x, ctrs, w):
    n = x.shape[0]
    bn = 1000 if n % 1000 == 0 else n
    grid = n // bn
    return pl.pallas_call(
        _proj_kernel,
        grid=(grid,),
        in_specs=[
            pl.BlockSpec((bn, D), lambda i: (i, 0)),
            pl.BlockSpec((bn, 2), lambda i: (i, 0)),
            pl.BlockSpec((D, D), lambda i: (0, 0)),
            pl.BlockSpec((2, D), lambda i: (0, 0)),
        ],
        out_specs=pl.BlockSpec((bn, D), lambda i: (i, 0)),
        out_shape=jax.ShapeDtypeStruct((n, D), jnp.float32),
    )(x, ctrs, w[:D], w[D:])


def _ctrproj_kernel(c_ref, wb_ref, o_ref):
    c = c_ref[...]
    o_ref[...] = c[:, 0:1] * wb_ref[0:1, :] + c[:, 1:2] * wb_ref[1:2, :]


def _ctrproj(ctrs, wb):
    n = ctrs.shape[0]
    bn = 1000 if n % 1000 == 0 else n
    grid = n // bn
    return pl.pallas_call(
        _ctrproj_kernel,
        grid=(grid,),
        in_specs=[
            pl.BlockSpec((bn, 2), lambda i: (i, 0)),
            pl.BlockSpec((2, D), lambda i: (0, 0)),
        ],
        out_specs=pl.BlockSpec((bn, D), lambda i: (i, 0)),
        out_shape=jax.ShapeDtypeStruct((n, D), jnp.float32),
    )(ctrs, wb)


def _map_update_kernel(n_ref, s0_ref, s1_ref, w1_ref, w2_ref, o_ref):
    o = jnp.dot(n_ref[...], w1_ref[...], preferred_element_type=jnp.float32)
    o += jnp.dot(s0_ref[...] + s1_ref[...], w2_ref[...],
                 preferred_element_type=jnp.float32)
    o_ref[...] = jnp.maximum(o, 0.0)


def _map_update(nodes, seg, w1, w2):
    n = nodes.shape[0]
    bn = 1000
    return pl.pallas_call(
        _map_update_kernel,
        grid=(n // bn,),
        in_specs=[
            pl.BlockSpec((bn, D), lambda i: (i, 0)),
            pl.BlockSpec((bn, D), lambda i: (i, 0)),
            pl.BlockSpec((bn, D), lambda i: (i, 0)),
            pl.BlockSpec((D, D), lambda i: (0, 0)),
            pl.BlockSpec((D, D), lambda i: (0, 0)),
        ],
        out_specs=pl.BlockSpec((bn, D), lambda i: (i, 0)),
        out_shape=jax.ShapeDtypeStruct((n, D), jnp.float32),
    )(nodes, seg[0], seg[1], w1, w2)


def _map_update2_kernel(n_ref, s0_ref, s1_ref, a0_ref, a1_ref, w1_ref, w2_ref,
                        o_ref):
    o = jnp.dot(n_ref[...], w1_ref[...], preferred_element_type=jnp.float32)
    o += jnp.dot(s0_ref[...] + s1_ref[...], w2_ref[...],
                 preferred_element_type=jnp.float32)
    o_ref[...] = jnp.maximum(o, 0.0) + a0_ref[...] + a1_ref[...]


def _map_update2(nodes, seg, a2m, w1, w2):
    n = nodes.shape[0]
    bn = 1000
    return pl.pallas_call(
        _map_update2_kernel,
        grid=(n // bn,),
        in_specs=[pl.BlockSpec((bn, D), lambda i: (i, 0))] * 5 + [
            pl.BlockSpec((D, D), lambda i: (0, 0)),
            pl.BlockSpec((D, D), lambda i: (0, 0)),
        ],
        out_specs=pl.BlockSpec((bn, D), lambda i: (i, 0)),
        out_shape=jax.ShapeDtypeStruct((n, D), jnp.float32),
    )(nodes, seg[0], seg[1], a2m[0], a2m[1], w1, w2)


def _pred_kernel(a_ref, p0_ref, p1_ref, wp_ref, bp_ref, o_ref):
    a5 = jnp.maximum(a_ref[...] + p0_ref[...] + p1_ref[...], 0.0)
    o_ref[...] = (jnp.dot(a5, wp_ref[...], preferred_element_type=jnp.float32)
                  + bp_ref[...])


def _pred(actors4, a2a, wp, bp):
    n = actors4.shape[0]
    m = wp.shape[1]
    return pl.pallas_call(
        _pred_kernel,
        grid=(1,),
        in_specs=[
            pl.BlockSpec((n, D), lambda i: (0, 0)),
            pl.BlockSpec((n, D), lambda i: (0, 0)),
            pl.BlockSpec((n, D), lambda i: (0, 0)),
            pl.BlockSpec((D, m), lambda i: (0, 0)),
            pl.BlockSpec((1, m), lambda i: (0, 0)),
        ],
        out_specs=pl.BlockSpec((n, m), lambda i: (0, 0)),
        out_shape=jax.ShapeDtypeStruct((n, m), jnp.float32),
    )(actors4, a2a[0], a2a[1], wp, bp)


def kernel(actors, actor_ctrs, nodes, node_ctrs, edge_index, a2m_src, a2m_dst,
           m2a_src, m2a_dst, a2a_index, rot, orig,
           W_map1, W_map2, W_a2m, W_m2a, W_a2a, W_pred, b_pred):
    src, dst = edge_index[0], edge_index[1]
    a2m_s, a2m_d = a2m_src, a2m_dst
    m2a_s, m2a_d = m2a_src, m2a_dst
    a2a_s, a2a_d = a2a_index[0], a2a_index[1]

    S_a = _proj(actors, actor_ctrs, W_a2m)
    Q_n = _ctrproj(node_ctrs, W_a2m[D:])

    seg1 = _seg_sum_nodes(nodes, src, dst)[:, :N_NODES]
    a2m_agg = _relu_agg_nodes(S_a, Q_n, a2m_s, a2m_d)[:, :N_NODES]
    nodes2 = _map_update2(nodes, seg1, a2m_agg, W_map1, W_map2)

    seg2 = _seg_sum_nodes(nodes2, src, dst)[:, :N_NODES]
    nodes3 = _map_update(nodes2, seg2, W_map1, W_map2)

    S_n = _proj(nodes3, node_ctrs, W_m2a)
    Q_a = _ctrproj(actor_ctrs, W_m2a[D:])
    m2a_agg = _relu_agg_actors(S_n, Q_a, m2a_s, m2a_d)[:, :N_ACTORS]
    actors4 = actors + m2a_agg[0] + m2a_agg[1]

    S_aa = _proj(actors4, actor_ctrs, W_a2a)
    Q_aa = _ctrproj(actor_ctrs, W_a2a[D:])
    a2a_agg = _relu_agg_a2a(S_aa, Q_aa, a2a_s, a2a_d)[:, :N_ACTORS]

    wp = (W_pred.reshape(D, -1, 2) @ rot).reshape(D, -1)
    bp = (b_pred.reshape(-1, 2) @ rot + orig[None, :]).reshape(1, -1)
    reg = _pred(actors4, a2a_agg, wp, bp)
    return reg.reshape(N_ACTORS, NUM_MODS, NUM_PREDS, 2)
